# baseline (device time: 318722 ns/iter reference)
import jax
import jax.numpy as jnp
from jax import lax
from jax.experimental import pallas as pl
from jax.experimental.pallas import tpu as pltpu

N_DEV = 4


def kernel(A, B):
    m, _ = A.shape
    _, n = B.shape
    chunk = m // N_DEV

    def body(a_ref, b_ref, out_ref, comm_ref, send_sems, recv_sems):
        my = lax.axis_index("i")
        left = lax.rem(my + N_DEV - 1, N_DEV)
        right = lax.rem(my + 1, N_DEV)

        barrier_sem = pltpu.get_barrier_semaphore()
        for nbr in (left, right):
            pl.semaphore_signal(
                barrier_sem, inc=1,
                device_id=(nbr,), device_id_type=pl.DeviceIdType.MESH,
            )
        pl.semaphore_wait(barrier_sem, 2)

        out_ref[...] = jnp.dot(
            a_ref[...], b_ref[...], preferred_element_type=jnp.float32
        )

        for s in range(N_DEV - 1):
            cs = lax.rem(my + N_DEV - s, N_DEV)
            rdma = pltpu.make_async_remote_copy(
                src_ref=out_ref.at[pl.ds(cs * chunk, chunk), :],
                dst_ref=comm_ref.at[s],
                send_sem=send_sems.at[s],
                recv_sem=recv_sems.at[s],
                device_id=(right,),
                device_id_type=pl.DeviceIdType.MESH,
            )
            rdma.start()
            rdma.wait()
            cr = lax.rem(my + N_DEV - s - 1, N_DEV)
            rows = pl.ds(cr * chunk, chunk)
            out_ref[rows, :] += comm_ref[s]

        for t in range(N_DEV - 1):
            ca = lax.rem(my + N_DEV + 1 - t, N_DEV)
            rows = pl.ds(ca * chunk, chunk)
            rdma = pltpu.make_async_remote_copy(
                src_ref=out_ref.at[rows, :],
                dst_ref=out_ref.at[rows, :],
                send_sem=send_sems.at[N_DEV - 1 + t],
                recv_sem=recv_sems.at[N_DEV - 1 + t],
                device_id=(right,),
                device_id_type=pl.DeviceIdType.MESH,
            )
            rdma.start()
            rdma.wait()

    return pl.pallas_call(
        body,
        out_shape=jax.ShapeDtypeStruct((m, n), jnp.float32),
        in_specs=[
            pl.BlockSpec(memory_space=pltpu.VMEM),
            pl.BlockSpec(memory_space=pltpu.VMEM),
        ],
        out_specs=pl.BlockSpec(memory_space=pltpu.VMEM),
        scratch_shapes=[
            pltpu.VMEM((N_DEV - 1, chunk, n), jnp.float32),
            pltpu.SemaphoreType.DMA((2 * (N_DEV - 1),)),
            pltpu.SemaphoreType.DMA((2 * (N_DEV - 1),)),
        ],
        compiler_params=pltpu.CompilerParams(collective_id=0),
    )(A, B)


# device time: 175887 ns/iter; 1.8121x vs baseline; 1.8121x over previous
import jax
import jax.numpy as jnp
from jax import lax
from jax.experimental import pallas as pl
from jax.experimental.pallas import tpu as pltpu

N_DEV = 4


def kernel(A, B):
    m, _ = A.shape
    _, n = B.shape
    hm, hn = m // 2, n // 2
    qm = m // 4

    def body(a_ref, b_ref, out_ref, c1h0, c1h1, c2h0, c2h1,
             send_sems, recv_sems):
        my = lax.axis_index("i")
        h1 = lax.bitwise_and(lax.bitwise_xor(my, lax.shift_right_logical(my, 1)), 1)
        h1p = lax.bitwise_and(lax.shift_right_logical(my, 1), 1)
        bit0 = lax.bitwise_and(my, 1)
        c0 = 2 * h1 + h1p
        p1 = lax.bitwise_xor(my, 1)
        p2 = 3 - my

        barrier_sem = pltpu.get_barrier_semaphore()
        for nbr in (p1, p2):
            pl.semaphore_signal(
                barrier_sem, inc=1,
                device_id=(nbr,), device_id_type=pl.DeviceIdType.MESH,
            )
        pl.semaphore_wait(barrier_sem, 2)

        def quadrant(r_off, c_off):
            out_ref[pl.ds(r_off, hm), pl.ds(c_off, hn)] = jnp.dot(
                a_ref[pl.ds(r_off, hm), :],
                b_ref[:, pl.ds(c_off, hn)],
                preferred_element_type=jnp.float32,
            )

        def exchange(src, dst, sem_idx, partner):
            return pltpu.make_async_remote_copy(
                src_ref=src, dst_ref=dst,
                send_sem=send_sems.at[sem_idx], recv_sem=recv_sems.at[sem_idx],
                device_id=(partner,), device_id_type=pl.DeviceIdType.MESH,
            )

        send_r0 = (1 - h1) * hm
        send_r1 = (1 - h1p) * hm
        quadrant(send_r0, 0)
        quadrant(send_r1, hn)
        rd10 = exchange(out_ref.at[pl.ds(send_r0, hm), pl.ds(0, hn)], c1h0, 0, p1)
        rd11 = exchange(out_ref.at[pl.ds(send_r1, hm), pl.ds(hn, hn)], c1h1, 1, p2)
        rd10.start()
        rd11.start()

        keep_r0 = h1 * hm
        keep_r1 = h1p * hm
        quadrant(keep_r0, 0)
        quadrant(keep_r1, hn)

        rd10.wait()
        out_ref[pl.ds(keep_r0, hm), pl.ds(0, hn)] += c1h0[...]
        rd11.wait()
        out_ref[pl.ds(keep_r1, hm), pl.ds(hn, hn)] += c1h1[...]

        s2_0 = (2 * h1 + 1 - h1p) * qm
        s2_1 = (2 * h1p + 1 - bit0) * qm
        rd20 = exchange(out_ref.at[pl.ds(s2_0, qm), pl.ds(0, hn)], c2h0, 2, p2)
        rd21 = exchange(out_ref.at[pl.ds(s2_1, qm), pl.ds(hn, hn)], c2h1, 3, p1)
        rd20.start()
        rd21.start()
        rd20.wait()
        out_ref[pl.ds(c0 * qm, qm), pl.ds(0, hn)] += c2h0[...]
        rd21.wait()
        out_ref[pl.ds(my * qm, qm), pl.ds(hn, hn)] += c2h1[...]

        rd30 = exchange(out_ref.at[pl.ds(c0 * qm, qm), pl.ds(0, hn)],
                        out_ref.at[pl.ds(c0 * qm, qm), pl.ds(0, hn)], 4, p2)
        rd31 = exchange(out_ref.at[pl.ds(my * qm, qm), pl.ds(hn, hn)],
                        out_ref.at[pl.ds(my * qm, qm), pl.ds(hn, hn)], 5, p1)
        rd30.start()
        rd31.start()
        rd30.wait()
        rd31.wait()

        rd40 = exchange(out_ref.at[pl.ds(h1 * hm, hm), pl.ds(0, hn)],
                        out_ref.at[pl.ds(h1 * hm, hm), pl.ds(0, hn)], 6, p1)
        rd41 = exchange(out_ref.at[pl.ds(h1p * hm, hm), pl.ds(hn, hn)],
                        out_ref.at[pl.ds(h1p * hm, hm), pl.ds(hn, hn)], 7, p2)
        rd40.start()
        rd41.start()
        rd40.wait()
        rd41.wait()

    return pl.pallas_call(
        body,
        out_shape=jax.ShapeDtypeStruct((m, n), jnp.float32),
        in_specs=[
            pl.BlockSpec(memory_space=pltpu.VMEM),
            pl.BlockSpec(memory_space=pltpu.VMEM),
        ],
        out_specs=pl.BlockSpec(memory_space=pltpu.VMEM),
        scratch_shapes=[
            pltpu.VMEM((hm, hn), jnp.float32),
            pltpu.VMEM((hm, hn), jnp.float32),
            pltpu.VMEM((qm, hn), jnp.float32),
            pltpu.VMEM((qm, hn), jnp.float32),
            pltpu.SemaphoreType.DMA((8,)),
            pltpu.SemaphoreType.DMA((8,)),
        ],
        compiler_params=pltpu.CompilerParams(collective_id=0),
    )(A, B)


# device time: 174257 ns/iter; 1.8290x vs baseline; 1.0094x over previous
import jax
import jax.numpy as jnp
from jax import lax
from jax.experimental import pallas as pl
from jax.experimental.pallas import tpu as pltpu

N_DEV = 4


def kernel(A, B):
    m, _ = A.shape
    _, n = B.shape
    hm, hn = m // 2, n // 2
    qm = m // 4

    def body(a_ref, b_ref, out_ref, q0, q1, c1h0, c1h1, c2h0, c2h1,
             send_sems, recv_sems):
        my = lax.axis_index("i")
        h1 = lax.bitwise_and(lax.bitwise_xor(my, lax.shift_right_logical(my, 1)), 1)
        h1p = lax.bitwise_and(lax.shift_right_logical(my, 1), 1)
        bit0 = lax.bitwise_and(my, 1)
        c0 = 2 * h1 + h1p
        p1 = lax.bitwise_xor(my, 1)
        p2 = 3 - my

        barrier_sem = pltpu.get_barrier_semaphore()
        for nbr in (p1, p2):
            pl.semaphore_signal(
                barrier_sem, inc=1,
                device_id=(nbr,), device_id_type=pl.DeviceIdType.MESH,
            )
        pl.semaphore_wait(barrier_sem, 2)

        def quadrant(r_off, c_off):
            out_ref[pl.ds(r_off, hm), pl.ds(c_off, hn)] = jnp.dot(
                a_ref[pl.ds(r_off, hm), :],
                b_ref[:, pl.ds(c_off, hn)],
                preferred_element_type=jnp.float32,
            )

        def exchange(src, dst, sem_idx, partner):
            return pltpu.make_async_remote_copy(
                src_ref=src, dst_ref=dst,
                send_sem=send_sems.at[sem_idx], recv_sem=recv_sems.at[sem_idx],
                device_id=(partner,), device_id_type=pl.DeviceIdType.MESH,
            )

        send_r0 = (1 - h1) * hm
        send_r1 = (1 - h1p) * hm
        q0[...] = jnp.dot(
            a_ref[pl.ds(send_r0, hm), :], b_ref[:, pl.ds(0, hn)],
            preferred_element_type=jnp.float32,
        )
        rd10 = exchange(q0, c1h0, 0, p1)
        rd10.start()
        q1[...] = jnp.dot(
            a_ref[pl.ds(send_r1, hm), :], b_ref[:, pl.ds(hn, hn)],
            preferred_element_type=jnp.float32,
        )
        rd11 = exchange(q1, c1h1, 1, p2)
        rd11.start()

        keep_r0 = h1 * hm
        keep_r1 = h1p * hm
        quadrant(keep_r0, 0)
        quadrant(keep_r1, hn)

        s2_0 = (2 * h1 + 1 - h1p) * qm
        s2_1 = (2 * h1p + 1 - bit0) * qm
        rd10.wait()
        out_ref[pl.ds(s2_0, qm), pl.ds(0, hn)] += c1h0[pl.ds((1 - h1p) * qm, qm), :]
        rd20 = exchange(out_ref.at[pl.ds(s2_0, qm), pl.ds(0, hn)], c2h0, 2, p2)
        rd20.start()
        rd11.wait()
        out_ref[pl.ds(s2_1, qm), pl.ds(hn, hn)] += c1h1[pl.ds((1 - bit0) * qm, qm), :]
        rd21 = exchange(out_ref.at[pl.ds(s2_1, qm), pl.ds(hn, hn)], c2h1, 3, p1)
        rd21.start()

        out_ref[pl.ds(c0 * qm, qm), pl.ds(0, hn)] += c1h0[pl.ds(h1p * qm, qm), :]
        out_ref[pl.ds(my * qm, qm), pl.ds(hn, hn)] += c1h1[pl.ds(bit0 * qm, qm), :]

        rd20.wait()
        out_ref[pl.ds(c0 * qm, qm), pl.ds(0, hn)] += c2h0[...]
        rd21.wait()
        out_ref[pl.ds(my * qm, qm), pl.ds(hn, hn)] += c2h1[...]

        rd30 = exchange(out_ref.at[pl.ds(c0 * qm, qm), pl.ds(0, hn)],
                        out_ref.at[pl.ds(c0 * qm, qm), pl.ds(0, hn)], 4, p2)
        rd31 = exchange(out_ref.at[pl.ds(my * qm, qm), pl.ds(hn, hn)],
                        out_ref.at[pl.ds(my * qm, qm), pl.ds(hn, hn)], 5, p1)
        rd30.start()
        rd31.start()
        rd30.wait()
        rd31.wait()

        rd40 = exchange(out_ref.at[pl.ds(h1 * hm, hm), pl.ds(0, hn)],
                        out_ref.at[pl.ds(h1 * hm, hm), pl.ds(0, hn)], 6, p1)
        rd41 = exchange(out_ref.at[pl.ds(h1p * hm, hm), pl.ds(hn, hn)],
                        out_ref.at[pl.ds(h1p * hm, hm), pl.ds(hn, hn)], 7, p2)
        rd40.start()
        rd41.start()
        rd40.wait()
        rd41.wait()

    return pl.pallas_call(
        body,
        out_shape=jax.ShapeDtypeStruct((m, n), jnp.float32),
        in_specs=[
            pl.BlockSpec(memory_space=pltpu.VMEM),
            pl.BlockSpec(memory_space=pltpu.VMEM),
        ],
        out_specs=pl.BlockSpec(memory_space=pltpu.VMEM),
        scratch_shapes=[
            pltpu.VMEM((hm, hn), jnp.float32),
            pltpu.VMEM((hm, hn), jnp.float32),
            pltpu.VMEM((hm, hn), jnp.float32),
            pltpu.VMEM((hm, hn), jnp.float32),
            pltpu.VMEM((qm, hn), jnp.float32),
            pltpu.VMEM((qm, hn), jnp.float32),
            pltpu.SemaphoreType.DMA((8,)),
            pltpu.SemaphoreType.DMA((8,)),
        ],
        compiler_params=pltpu.CompilerParams(
            collective_id=0,
            vmem_limit_bytes=60 * 1024 * 1024,
        ),
    )(A, B)


# device time: 168637 ns/iter; 1.8900x vs baseline; 1.0333x over previous
import jax
import jax.numpy as jnp
from jax import lax
from jax.experimental import pallas as pl
from jax.experimental.pallas import tpu as pltpu

N_DEV = 4


def kernel(A, B):
    m, _ = A.shape
    _, n = B.shape
    hm, hn = m // 2, n // 2
    qm = m // 4

    def body(a_ref, b_ref, out_ref, q0, q1, c1h0, c1h1, c2h0, c2h1,
             send_sems, recv_sems):
        my = lax.axis_index("i")
        h1 = lax.bitwise_and(lax.bitwise_xor(my, lax.shift_right_logical(my, 1)), 1)
        h1p = lax.bitwise_and(lax.shift_right_logical(my, 1), 1)
        bit0 = lax.bitwise_and(my, 1)
        c0 = 2 * h1 + h1p
        p1 = lax.bitwise_xor(my, 1)
        p2 = 3 - my

        barrier_sem = pltpu.get_barrier_semaphore()
        for nbr in (p1, p2):
            pl.semaphore_signal(
                barrier_sem, inc=1,
                device_id=(nbr,), device_id_type=pl.DeviceIdType.MESH,
            )
        pl.semaphore_wait(barrier_sem, 2)

        def exchange(src, dst, sem_idx, partner):
            return pltpu.make_async_remote_copy(
                src_ref=src, dst_ref=dst,
                send_sem=send_sems.at[sem_idx], recv_sem=recv_sems.at[sem_idx],
                device_id=(partner,), device_id_type=pl.DeviceIdType.MESH,
            )

        send_r0 = (1 - h1) * hm
        send_r1 = (1 - h1p) * hm
        off0a = (1 - h1p) * qm
        off0b = h1p * qm
        off1a = bit0 * qm
        off1b = (1 - bit0) * qm

        def half_dot(dst_ref, dst_off, a_row, b_col):
            dst_ref[pl.ds(dst_off, qm), :] = jnp.dot(
                a_ref[pl.ds(a_row, qm), :],
                b_ref[:, pl.ds(b_col, hn)],
                preferred_element_type=jnp.float32,
            )

        half_dot(q0, off0a, send_r0 + off0a, 0)
        rd1a = exchange(q0.at[pl.ds(off0a, qm), :], c1h0.at[0], 0, p1)
        rd1a.start()
        half_dot(q1, off1a, send_r1 + off1a, hn)
        rd2a = exchange(q1.at[pl.ds(off1a, qm), :], c1h1.at[0], 1, p2)
        rd2a.start()
        half_dot(q0, off0b, send_r0 + off0b, 0)
        rd1b = exchange(q0.at[pl.ds(off0b, qm), :], c1h0.at[1], 2, p1)
        rd1b.start()
        half_dot(q1, off1b, send_r1 + off1b, hn)
        rd2b = exchange(q1.at[pl.ds(off1b, qm), :], c1h1.at[1], 3, p2)
        rd2b.start()

        def quadrant(r_off, c_off):
            out_ref[pl.ds(r_off, hm), pl.ds(c_off, hn)] = jnp.dot(
                a_ref[pl.ds(r_off, hm), :],
                b_ref[:, pl.ds(c_off, hn)],
                preferred_element_type=jnp.float32,
            )

        quadrant(h1 * hm, 0)
        quadrant(h1p * hm, hn)

        s2_0 = (2 * h1 + 1 - h1p) * qm
        s2_1 = (2 * h1p + 1 - bit0) * qm
        rd1a.wait()
        out_ref[pl.ds(s2_0, qm), pl.ds(0, hn)] += c1h0[0]
        rd20 = exchange(out_ref.at[pl.ds(s2_0, qm), pl.ds(0, hn)], c2h0, 4, p2)
        rd20.start()
        rd2a.wait()
        out_ref[pl.ds(s2_1, qm), pl.ds(hn, hn)] += c1h1[0]
        rd21 = exchange(out_ref.at[pl.ds(s2_1, qm), pl.ds(hn, hn)], c2h1, 5, p1)
        rd21.start()

        rd1b.wait()
        out_ref[pl.ds(c0 * qm, qm), pl.ds(0, hn)] += c1h0[1]
        rd2b.wait()
        out_ref[pl.ds(my * qm, qm), pl.ds(hn, hn)] += c1h1[1]

        rd20.wait()
        out_ref[pl.ds(c0 * qm, qm), pl.ds(0, hn)] += c2h0[...]
        rd30 = exchange(out_ref.at[pl.ds(c0 * qm, qm), pl.ds(0, hn)],
                        out_ref.at[pl.ds(c0 * qm, qm), pl.ds(0, hn)], 6, p2)
        rd30.start()
        rd21.wait()
        out_ref[pl.ds(my * qm, qm), pl.ds(hn, hn)] += c2h1[...]
        rd31 = exchange(out_ref.at[pl.ds(my * qm, qm), pl.ds(hn, hn)],
                        out_ref.at[pl.ds(my * qm, qm), pl.ds(hn, hn)], 7, p1)
        rd31.start()

        rd4a0 = exchange(out_ref.at[pl.ds(c0 * qm, qm), pl.ds(0, hn)],
                         out_ref.at[pl.ds(c0 * qm, qm), pl.ds(0, hn)], 8, p1)
        rd4a0.start()
        rd4a1 = exchange(out_ref.at[pl.ds(my * qm, qm), pl.ds(hn, hn)],
                         out_ref.at[pl.ds(my * qm, qm), pl.ds(hn, hn)], 9, p2)
        rd4a1.start()

        rd30.wait()
        rd4b0 = exchange(out_ref.at[pl.ds(s2_0, qm), pl.ds(0, hn)],
                         out_ref.at[pl.ds(s2_0, qm), pl.ds(0, hn)], 10, p1)
        rd4b0.start()
        rd31.wait()
        rd4b1 = exchange(out_ref.at[pl.ds(s2_1, qm), pl.ds(hn, hn)],
                         out_ref.at[pl.ds(s2_1, qm), pl.ds(hn, hn)], 11, p2)
        rd4b1.start()

        rd4a0.wait()
        rd4a1.wait()
        rd4b0.wait()
        rd4b1.wait()

    return pl.pallas_call(
        body,
        out_shape=jax.ShapeDtypeStruct((m, n), jnp.float32),
        in_specs=[
            pl.BlockSpec(memory_space=pltpu.VMEM),
            pl.BlockSpec(memory_space=pltpu.VMEM),
        ],
        out_specs=pl.BlockSpec(memory_space=pltpu.VMEM),
        scratch_shapes=[
            pltpu.VMEM((hm, hn), jnp.float32),
            pltpu.VMEM((hm, hn), jnp.float32),
            pltpu.VMEM((2, qm, hn), jnp.float32),
            pltpu.VMEM((2, qm, hn), jnp.float32),
            pltpu.VMEM((qm, hn), jnp.float32),
            pltpu.VMEM((qm, hn), jnp.float32),
            pltpu.SemaphoreType.DMA((12,)),
            pltpu.SemaphoreType.DMA((12,)),
        ],
        compiler_params=pltpu.CompilerParams(
            collective_id=0,
            vmem_limit_bytes=60 * 1024 * 1024,
        ),
    )(A, B)


# device time: 163017 ns/iter; 1.9551x vs baseline; 1.0345x over previous
import jax
import jax.numpy as jnp
from jax import lax
from jax.experimental import pallas as pl
from jax.experimental.pallas import tpu as pltpu

N_DEV = 4


def kernel(A, B):
    m, _ = A.shape
    _, n = B.shape
    hm, hn = m // 2, n // 2
    qm = m // 4

    def body(a_ref, b_ref, out_ref, q0, q1, k0, k1, c1h0, c1h1, c2h0, c2h1,
             send_sems, recv_sems, copy_sems):
        my = lax.axis_index("i")
        h1 = lax.bitwise_and(lax.bitwise_xor(my, lax.shift_right_logical(my, 1)), 1)
        h1p = lax.bitwise_and(lax.shift_right_logical(my, 1), 1)
        bit0 = lax.bitwise_and(my, 1)
        c0 = 2 * h1 + h1p
        p1 = lax.bitwise_xor(my, 1)
        p2 = 3 - my

        barrier_sem = pltpu.get_barrier_semaphore()
        for nbr in (p1, p2):
            pl.semaphore_signal(
                barrier_sem, inc=1,
                device_id=(nbr,), device_id_type=pl.DeviceIdType.MESH,
            )
        pl.semaphore_wait(barrier_sem, 2)

        def exchange(src, dst, sem_idx, partner):
            return pltpu.make_async_remote_copy(
                src_ref=src, dst_ref=dst,
                send_sem=send_sems.at[sem_idx], recv_sem=recv_sems.at[sem_idx],
                device_id=(partner,), device_id_type=pl.DeviceIdType.MESH,
            )

        send_r0 = (1 - h1) * hm
        send_r1 = (1 - h1p) * hm
        off0a = (1 - h1p) * qm
        off0b = h1p * qm
        off1a = bit0 * qm
        off1b = (1 - bit0) * qm

        def half_dot(dst_ref, dst_off, a_row, b_col):
            dst_ref[pl.ds(dst_off, qm), :] = jnp.dot(
                a_ref[pl.ds(a_row, qm), :],
                b_ref[:, pl.ds(b_col, hn)],
                preferred_element_type=jnp.float32,
            )

        half_dot(q0, off0a, send_r0 + off0a, 0)
        rd1a = exchange(q0.at[pl.ds(off0a, qm), :], c1h0.at[0], 0, p1)
        rd1a.start()
        half_dot(q1, off1a, send_r1 + off1a, hn)
        rd2a = exchange(q1.at[pl.ds(off1a, qm), :], c1h1.at[0], 1, p2)
        rd2a.start()
        half_dot(q0, off0b, send_r0 + off0b, 0)
        rd1b = exchange(q0.at[pl.ds(off0b, qm), :], c1h0.at[1], 2, p1)
        rd1b.start()
        half_dot(q1, off1b, send_r1 + off1b, hn)
        rd2b = exchange(q1.at[pl.ds(off1b, qm), :], c1h1.at[1], 3, p2)
        rd2b.start()

        keep_r0 = h1 * hm
        keep_r1 = h1p * hm
        k0[...] = jnp.dot(
            a_ref[pl.ds(keep_r0, hm), :], b_ref[:, pl.ds(0, hn)],
            preferred_element_type=jnp.float32,
        )
        k1[...] = jnp.dot(
            a_ref[pl.ds(keep_r1, hm), :], b_ref[:, pl.ds(hn, hn)],
            preferred_element_type=jnp.float32,
        )

        f0 = (1 - h1p) * qm
        o0 = h1p * qm
        f1 = (1 - bit0) * qm
        o1 = bit0 * qm

        rd1a.wait()
        k0[pl.ds(f0, qm), :] += c1h0[0]
        rd20 = exchange(k0.at[pl.ds(f0, qm), :], c2h0, 4, p2)
        rd20.start()
        rd2a.wait()
        k1[pl.ds(f1, qm), :] += c1h1[0]
        rd21 = exchange(k1.at[pl.ds(f1, qm), :], c2h1, 5, p1)
        rd21.start()

        rd1b.wait()
        k0[pl.ds(o0, qm), :] += c1h0[1]
        rd2b.wait()
        k1[pl.ds(o1, qm), :] += c1h1[1]

        rd20.wait()
        k0[pl.ds(o0, qm), :] += c2h0[...]
        rd30 = exchange(k0.at[pl.ds(o0, qm), :], k0.at[pl.ds(o0, qm), :], 6, p2)
        rd30.start()
        rd21.wait()
        k1[pl.ds(o1, qm), :] += c2h1[...]
        rd31 = exchange(k1.at[pl.ds(o1, qm), :], k1.at[pl.ds(o1, qm), :], 7, p1)
        rd31.start()

        rd4a0 = exchange(k0.at[pl.ds(o0, qm), :],
                         out_ref.at[pl.ds(c0 * qm, qm), pl.ds(0, hn)], 8, p1)
        rd4a0.start()
        rd4a1 = exchange(k1.at[pl.ds(o1, qm), :],
                         out_ref.at[pl.ds(my * qm, qm), pl.ds(hn, hn)], 9, p2)
        rd4a1.start()

        s2_0 = (2 * h1 + 1 - h1p) * qm
        s2_1 = (2 * h1p + 1 - bit0) * qm
        rd30.wait()
        rd4b0 = exchange(k0.at[pl.ds(f0, qm), :],
                         out_ref.at[pl.ds(s2_0, qm), pl.ds(0, hn)], 10, p1)
        rd4b0.start()
        cp0 = pltpu.make_async_copy(
            k0, out_ref.at[pl.ds(keep_r0, hm), pl.ds(0, hn)], copy_sems.at[0]
        )
        cp0.start()
        rd31.wait()
        rd4b1 = exchange(k1.at[pl.ds(f1, qm), :],
                         out_ref.at[pl.ds(s2_1, qm), pl.ds(hn, hn)], 11, p2)
        rd4b1.start()
        cp1 = pltpu.make_async_copy(
            k1, out_ref.at[pl.ds(keep_r1, hm), pl.ds(hn, hn)], copy_sems.at[1]
        )
        cp1.start()

        rd4a0.wait()
        rd4a1.wait()
        rd4b0.wait()
        rd4b1.wait()
        cp0.wait()
        cp1.wait()

    return pl.pallas_call(
        body,
        out_shape=jax.ShapeDtypeStruct((m, n), jnp.float32),
        in_specs=[
            pl.BlockSpec(memory_space=pltpu.VMEM),
            pl.BlockSpec(memory_space=pltpu.VMEM),
        ],
        out_specs=pl.BlockSpec(memory_space=pl.ANY),
        scratch_shapes=[
            pltpu.VMEM((hm, hn), jnp.float32),
            pltpu.VMEM((hm, hn), jnp.float32),
            pltpu.VMEM((hm, hn), jnp.float32),
            pltpu.VMEM((hm, hn), jnp.float32),
            pltpu.VMEM((2, qm, hn), jnp.float32),
            pltpu.VMEM((2, qm, hn), jnp.float32),
            pltpu.VMEM((qm, hn), jnp.float32),
            pltpu.VMEM((qm, hn), jnp.float32),
            pltpu.SemaphoreType.DMA((12,)),
            pltpu.SemaphoreType.DMA((12,)),
            pltpu.SemaphoreType.DMA((2,)),
        ],
        compiler_params=pltpu.CompilerParams(
            collective_id=0,
            vmem_limit_bytes=60 * 1024 * 1024,
        ),
    )(A, B)
